# Initial kernel scaffold; baseline (speedup 1.0000x reference)
import jax
import jax.numpy as jnp
from jax.experimental import pallas as pl


def kernel(static_feats, dynamic_state, edge_index, pos, w_ne, b_ne, w_ee, b_ee, w_msg, b_msg, w_upd, b_upd, w_dec, b_dec):
    raise NotImplementedError("write your pallas kernel here")


# speedup vs baseline: 1.4935x; 1.4935x over previous
import functools

import jax
import jax.numpy as jnp
from jax.experimental import pallas as pl
from jax.experimental.pallas import tpu as pltpu

N = 100000
E = 1600000
H = 32

_BLK = 2000


def _node_pre_body(sf, ds, pos, w_ne, b_ne, w1, w2b, hn_o, ts_o, td_o):
    nf = jnp.concatenate([sf[...], ds[...]], axis=1)
    hn = jnp.maximum(nf @ w_ne[...] + b_ne[...], 0.0)
    hn_o[...] = hn
    a = hn @ w1[...]
    b = hn @ w2b[...]
    p = pos[...]
    pad = jnp.zeros((p.shape[0], 14), jnp.float32)
    ts_o[...] = jnp.concatenate([a, p, pad], axis=1)
    td_o[...] = jnp.concatenate([b, p, pad], axis=1)


def _node_pre(sf, ds, pos, w_ne, b_ne, w1, w2b):
    grid = (N // _BLK,)
    row = lambda i: (i, 0)
    return pl.pallas_call(
        _node_pre_body,
        grid=grid,
        in_specs=[
            pl.BlockSpec((_BLK, 2), row),
            pl.BlockSpec((_BLK, 2), row),
            pl.BlockSpec((_BLK, 2), row),
            pl.BlockSpec((4, H), lambda i: (0, 0)),
            pl.BlockSpec((H,), lambda i: (0,)),
            pl.BlockSpec((H, H), lambda i: (0, 0)),
            pl.BlockSpec((H, H), lambda i: (0, 0)),
        ],
        out_specs=[
            pl.BlockSpec((_BLK, H), row),
            pl.BlockSpec((_BLK, 48), row),
            pl.BlockSpec((_BLK, 48), row),
        ],
        out_shape=[
            jax.ShapeDtypeStruct((N, H), jnp.float32),
            jax.ShapeDtypeStruct((N, 48), jnp.float32),
            jax.ShapeDtypeStruct((N, 48), jnp.float32),
        ],
    )(sf, ds, pos, w_ne, b_ne, w1, w2b)


def _update_body(hn, agg, ds, w_upd, b_upd, w_dec, b_dec, out_o):
    z = jnp.concatenate([hn[...], agg[...]], axis=1)
    hu = jnp.maximum(z @ w_upd[...] + b_upd[...], 0.0)
    out_o[...] = ds[...] + hu @ w_dec[...] + b_dec[...]


def _update(hn, agg, ds, w_upd, b_upd, w_dec, b_dec):
    grid = (N // _BLK,)
    row = lambda i: (i, 0)
    return pl.pallas_call(
        _update_body,
        grid=grid,
        in_specs=[
            pl.BlockSpec((_BLK, H), row),
            pl.BlockSpec((_BLK, H), row),
            pl.BlockSpec((_BLK, 2), row),
            pl.BlockSpec((2 * H, H), lambda i: (0, 0)),
            pl.BlockSpec((H,), lambda i: (0,)),
            pl.BlockSpec((H, 2), lambda i: (0, 0)),
            pl.BlockSpec((2,), lambda i: (0,)),
        ],
        out_specs=pl.BlockSpec((_BLK, 2), row),
        out_shape=jax.ShapeDtypeStruct((N, 2), jnp.float32),
    )(hn, agg, ds, w_upd, b_upd, w_dec, b_dec)


def _edge_stage_jnp(t_src, t_dst, src, dst, w_ee, b_ee, w3, b_msg):
    gs = jnp.take(t_src, src, axis=0)
    gd = jnp.take(t_dst, dst, axis=0)
    rel = gd[:, 32:34] - gs[:, 32:34]
    dist = jnp.sqrt(jnp.sum(rel * rel, axis=1, keepdims=True) + 1e-12)
    ef = jnp.concatenate([rel, dist], axis=1)
    he = jnp.maximum(ef @ w_ee + b_ee, 0.0)
    m = jnp.maximum(gs[:, :32] + gd[:, :32] + he @ w3 + b_msg, 0.0)
    return jax.ops.segment_sum(m, dst, num_segments=N)


def kernel(static_feats, dynamic_state, edge_index, pos,
           w_ne, b_ne, w_ee, b_ee, w_msg, b_msg, w_upd, b_upd, w_dec, b_dec):
    w1 = w_msg[:H]
    w2 = w_msg[H:2 * H]
    w3 = w_msg[2 * H:]
    hn, t_src, t_dst = _node_pre(static_feats, dynamic_state, pos,
                                 w_ne, b_ne, w1, w2)
    src = edge_index[0]
    dst = edge_index[1]
    agg = _edge_stage_jnp(t_src, t_dst, src, dst, w_ee, b_ee, w3, b_msg)
    return _update(hn, agg, dynamic_state, w_upd, b_upd, w_dec, b_dec)


# speedup vs baseline: 3.4561x; 2.3142x over previous
import functools

import jax
import jax.numpy as jnp
from jax import lax
from jax.experimental import pallas as pl
from jax.experimental.pallas import tpu as pltpu
from jax.experimental.pallas import tpu_sc as plsc

N = 100000
E = 1600000
H = 32
HH = 16

NC = 2
NS = 16
L = 16

_BLK = 2000
_BLKE = 3200

CE = 512
NB = CE // 128
NCHUNK = E // CE
CE1 = 512
NB1 = CE1 // 128
NCHUNK1 = E // CE1
NPAD = 100096
ZSL = NPAD // NS


def _node_pre_body(sf, ds, w_ne, b_ne, w1, w2, b_msg,
                   hn_o, a0_o, a1_o, b0_o, b1_o):
    nf = jnp.concatenate([sf[...], ds[...]], axis=1)
    hn = jnp.maximum(nf @ w_ne[...] + b_ne[...], 0.0)
    hn_o[...] = hn
    a = hn @ w1[...]
    b = hn @ w2[...] + b_msg[...]
    a0_o[...] = a[:, :HH]
    a1_o[...] = a[:, HH:]
    b0_o[...] = b[:, :HH]
    b1_o[...] = b[:, HH:]


def _node_pre(sf, ds, w_ne, b_ne, w1, w2, b_msg):
    row = lambda i: (i, 0)
    full = jax.ShapeDtypeStruct((N, H), jnp.float32)
    half = jax.ShapeDtypeStruct((N, HH), jnp.float32)
    return pl.pallas_call(
        _node_pre_body,
        grid=(N // _BLK,),
        in_specs=[
            pl.BlockSpec((_BLK, 2), row),
            pl.BlockSpec((_BLK, 2), row),
            pl.BlockSpec((4, H), lambda i: (0, 0)),
            pl.BlockSpec((H,), lambda i: (0,)),
            pl.BlockSpec((H, H), lambda i: (0, 0)),
            pl.BlockSpec((H, H), lambda i: (0, 0)),
            pl.BlockSpec((H,), lambda i: (0,)),
        ],
        out_specs=[
            pl.BlockSpec((_BLK, H), row),
            pl.BlockSpec((_BLK, HH), row),
            pl.BlockSpec((_BLK, HH), row),
            pl.BlockSpec((_BLK, HH), row),
            pl.BlockSpec((_BLK, HH), row),
        ],
        out_shape=[full, half, half, half, half],
    )(sf, ds, w_ne, b_ne, w1, w2, b_msg)


def _edge_enc_body(rx, ry, dd, w_ee, b_ee, w3, clo_o, chi_o):
    w = w_ee[...]
    he = jnp.maximum(rx[...] * w[0] + ry[...] * w[1] + dd[...] * w[2]
                     + b_ee[...], 0.0)
    c = he @ w3[...]
    clo_o[...] = c[:, :HH]
    chi_o[...] = c[:, HH:]


def _edge_enc(rx, ry, dd, w_ee, b_ee, w3):
    row = lambda i: (i, 0)
    half = jax.ShapeDtypeStruct((E, HH), jnp.float32)
    return pl.pallas_call(
        _edge_enc_body,
        grid=(E // _BLKE,),
        in_specs=[
            pl.BlockSpec((_BLKE, 1), row),
            pl.BlockSpec((_BLKE, 1), row),
            pl.BlockSpec((_BLKE, 1), row),
            pl.BlockSpec((3, H), lambda i: (0, 0)),
            pl.BlockSpec((H,), lambda i: (0,)),
            pl.BlockSpec((H, H), lambda i: (0, 0)),
        ],
        out_specs=[
            pl.BlockSpec((_BLKE, HH), row),
            pl.BlockSpec((_BLKE, HH), row),
        ],
        out_shape=[half, half],
    )(rx, ry, dd, w_ee, b_ee, w3)


def _update_body(hn, agg0, agg1, ds, w_upd, b_upd, w_dec, b_dec, out_o):
    z = jnp.concatenate([hn[...], agg0[...], agg1[...]], axis=1)
    hu = jnp.maximum(z @ w_upd[...] + b_upd[...], 0.0)
    out_o[...] = ds[...] + hu @ w_dec[...] + b_dec[...]


def _update(hn, agg0, agg1, ds, w_upd, b_upd, w_dec, b_dec):
    row = lambda i: (i, 0)
    return pl.pallas_call(
        _update_body,
        grid=(N // _BLK,),
        in_specs=[
            pl.BlockSpec((_BLK, H), row),
            pl.BlockSpec((_BLK, HH), row),
            pl.BlockSpec((_BLK, HH), row),
            pl.BlockSpec((_BLK, 2), row),
            pl.BlockSpec((2 * H, H), lambda i: (0, 0)),
            pl.BlockSpec((H,), lambda i: (0,)),
            pl.BlockSpec((H, 2), lambda i: (0, 0)),
            pl.BlockSpec((2,), lambda i: (0,)),
        ],
        out_specs=pl.BlockSpec((_BLK, 2), row),
        out_shape=jax.ShapeDtypeStruct((N, 2), jnp.float32),
    )(hn, agg0, agg1, ds, w_upd, b_upd, w_dec, b_dec)


def _sqrt16(x):
    i = lax.bitcast_convert_type(x, jnp.int32)
    y = lax.bitcast_convert_type(
        lax.shift_right_logical(i, 1) + jnp.int32(0x1FBD1DF5), jnp.float32)
    y = 0.5 * (y + x / y)
    y = 0.5 * (y + x / y)
    y = 0.5 * (y + x / y)
    return y


def _geom_body(src_h, dst_h, px_h, py_h, rx_h, ry_h, dd_h,
               s_buf, d_buf, xs, ys, xd, yd, rxb, ryb, ddb, sem):
    wid = lax.axis_index("s") * NC + lax.axis_index("c")

    def chunk(i, _):
        t = wid + i * (NC * NS)

        @pl.when(t < NCHUNK1)
        def _():
            pltpu.sync_copy(src_h.at[t], s_buf)
            pltpu.sync_copy(dst_h.at[t], d_buf)
            for b in range(NB1):
                sl = pl.ds(b * 128, 128)
                pltpu.async_copy(px_h.at[s_buf.at[b]], xs.at[sl], sem)
                pltpu.async_copy(py_h.at[s_buf.at[b]], ys.at[sl], sem)
                pltpu.async_copy(px_h.at[d_buf.at[b]], xd.at[sl], sem)
                pltpu.async_copy(py_h.at[d_buf.at[b]], yd.at[sl], sem)
            for b in range(NB1):
                sl = pl.ds(b * 128, 128)
                pltpu.make_async_copy(px_h.at[s_buf.at[b]], xs.at[sl], sem).wait()
                pltpu.make_async_copy(py_h.at[s_buf.at[b]], ys.at[sl], sem).wait()
                pltpu.make_async_copy(px_h.at[d_buf.at[b]], xd.at[sl], sem).wait()
                pltpu.make_async_copy(py_h.at[d_buf.at[b]], yd.at[sl], sem).wait()

            def grp(g, _):
                sl = pl.ds(g * L, L)
                rx = xd[sl] - xs[sl]
                ry = yd[sl] - ys[sl]
                d2 = rx * rx + ry * ry + 1e-12
                rxb[sl] = rx
                ryb[sl] = ry
                ddb[sl] = _sqrt16(d2)
                return 0
            lax.fori_loop(0, CE1 // L, grp, 0)

            base = t * CE1
            pltpu.sync_copy(rxb, rx_h.at[pl.ds(base, CE1)])
            pltpu.sync_copy(ryb, ry_h.at[pl.ds(base, CE1)])
            pltpu.sync_copy(ddb, dd_h.at[pl.ds(base, CE1)])
        return 0

    lax.fori_loop(0, (NCHUNK1 + NC * NS - 1) // (NC * NS), chunk, 0)


def _edge_geom(src2d, dst2d, px, py):
    mesh = plsc.VectorSubcoreMesh(core_axis_name="c", subcore_axis_name="s")
    f = functools.partial(
        pl.kernel,
        out_type=[
            jax.ShapeDtypeStruct((E,), jnp.float32),
            jax.ShapeDtypeStruct((E,), jnp.float32),
            jax.ShapeDtypeStruct((E,), jnp.float32),
        ],
        mesh=mesh,
        scratch_types=[
            pltpu.VMEM((NB1, 128), jnp.int32),
            pltpu.VMEM((NB1, 128), jnp.int32),
            pltpu.VMEM((CE1,), jnp.float32),
            pltpu.VMEM((CE1,), jnp.float32),
            pltpu.VMEM((CE1,), jnp.float32),
            pltpu.VMEM((CE1,), jnp.float32),
            pltpu.VMEM((CE1,), jnp.float32),
            pltpu.VMEM((CE1,), jnp.float32),
            pltpu.VMEM((CE1,), jnp.float32),
            pltpu.SemaphoreType.DMA,
        ],
    )(_geom_body)
    return f(src2d, dst2d, px, py)


def _agg_body(src_h, dst_h, a0_h, a1_h, b0_h, b1_h, clo_h, chi_h,
              out0_h, out1_h,
              s_buf, d_buf, ga, gb, gc, agg_sh, gsem, ssem):
    cid = lax.axis_index("c")
    sid = lax.axis_index("s")

    def zrow(r, _):
        gc[r, pl.ds(0, HH)] = jnp.zeros((HH,), jnp.float32)
        return 0
    lax.fori_loop(0, CE, zrow, 0)
    zbase = sid * ZSL
    done = 0
    while done < ZSL:
        step = min(CE, ZSL - done)
        pltpu.sync_copy(gc.at[pl.ds(0, step)],
                        agg_sh.at[pl.ds(zbase + done, step)])
        done += step
    plsc.subcore_barrier()

    def run(a_h, b_h, c_h):
        def chunk(i, _):
            t = sid + i * NS

            @pl.when(t < NCHUNK)
            def _():
                base = t * CE
                pltpu.sync_copy(src_h.at[t], s_buf)
                pltpu.sync_copy(dst_h.at[t], d_buf)
                for b in range(NB):
                    sl = pl.ds(b * 128, 128)
                    pltpu.async_copy(a_h.at[s_buf.at[b]], ga.at[sl], gsem)
                    pltpu.async_copy(b_h.at[d_buf.at[b]], gb.at[sl], gsem)
                pltpu.async_copy(c_h.at[pl.ds(base, CE)], gc, gsem)
                for b in range(NB):
                    sl = pl.ds(b * 128, 128)
                    pltpu.make_async_copy(a_h.at[s_buf.at[b]], ga.at[sl], gsem).wait()
                    pltpu.make_async_copy(b_h.at[d_buf.at[b]], gb.at[sl], gsem).wait()
                pltpu.make_async_copy(c_h.at[pl.ds(base, CE)], gc, gsem).wait()

                def crow(r, _):
                    sl8 = pl.ds(r * 8, 8)
                    gc[sl8, :] = jnp.maximum(
                        ga[sl8, :] + gb[sl8, :] + gc[sl8, :], 0.0)
                    return 0
                lax.fori_loop(0, CE // 8, crow, 0)

                for b in range(NB):
                    sl = pl.ds(b * 128, 128)
                    pltpu.async_copy(
                        gc.at[sl], agg_sh.at[d_buf.at[b]], ssem, add=True)
                for b in range(NB):
                    sl = pl.ds(b * 128, 128)
                    pltpu.make_async_copy(
                        gc.at[sl], agg_sh.at[d_buf.at[b]], ssem).wait()
            return 0

        lax.fori_loop(0, (NCHUNK + NS - 1) // NS, chunk, 0)

    @pl.when(cid == 0)
    def _():
        run(a0_h, b0_h, clo_h)

    @pl.when(cid == 1)
    def _():
        run(a1_h, b1_h, chi_h)

    plsc.subcore_barrier()
    obase = sid * ZSL

    @pl.when(cid == 0)
    def _():
        pltpu.sync_copy(agg_sh.at[pl.ds(obase, ZSL)],
                        out0_h.at[pl.ds(obase, ZSL)])

    @pl.when(cid == 1)
    def _():
        pltpu.sync_copy(agg_sh.at[pl.ds(obase, ZSL)],
                        out1_h.at[pl.ds(obase, ZSL)])


def _edge_agg(src3d, dst3d, a0, a1, b0, b1, clo, chi):
    mesh = plsc.VectorSubcoreMesh(core_axis_name="c", subcore_axis_name="s")
    half = jax.ShapeDtypeStruct((NPAD, HH), jnp.float32)
    f = functools.partial(
        pl.kernel,
        out_type=[half, half],
        mesh=mesh,
        compiler_params=pltpu.CompilerParams(use_tc_tiling_on_sc=False),
        scratch_types=[
            pltpu.VMEM((NB, 128), jnp.int32),
            pltpu.VMEM((NB, 128), jnp.int32),
            pltpu.VMEM((CE, HH), jnp.float32),
            pltpu.VMEM((CE, HH), jnp.float32),
            pltpu.VMEM((CE, HH), jnp.float32),
            pltpu.VMEM_SHARED((NPAD, HH), jnp.float32),
            pltpu.SemaphoreType.DMA,
            pltpu.SemaphoreType.DMA,
        ],
    )(_agg_body)
    return f(src3d, dst3d, a0, a1, b0, b1, clo, chi)


def kernel(static_feats, dynamic_state, edge_index, pos,
           w_ne, b_ne, w_ee, b_ee, w_msg, b_msg, w_upd, b_upd, w_dec, b_dec):
    src3d_1 = edge_index[0].reshape(NCHUNK1, NB1, 128)
    dst3d_1 = edge_index[1].reshape(NCHUNK1, NB1, 128)
    src3d_2 = edge_index[0].reshape(NCHUNK, NB, 128)
    dst3d_2 = edge_index[1].reshape(NCHUNK, NB, 128)
    w1 = w_msg[:H]
    w2 = w_msg[H:2 * H]
    w3 = w_msg[2 * H:]
    hn, a0, a1, b0, b1 = _node_pre(
        static_feats, dynamic_state, w_ne, b_ne, w1, w2, b_msg)
    px = pos[:, 0] + 0.0
    py = pos[:, 1] + 0.0
    rx, ry, dd = _edge_geom(src3d_1, dst3d_1, px, py)
    clo, chi = _edge_enc(rx.reshape(E, 1), ry.reshape(E, 1), dd.reshape(E, 1),
                         w_ee, b_ee, w3)
    agg0, agg1 = _edge_agg(src3d_2, dst3d_2, a0, a1, b0, b1, clo, chi)
    return _update(hn, agg0, agg1, dynamic_state, w_upd, b_upd, w_dec, b_dec)
